# baseline (speedup 1.0000x reference)
import functools

import jax
import jax.numpy as jnp
from jax import lax
from jax.experimental import pallas as pl
from jax.experimental.pallas import tpu as pltpu
from jax.experimental.pallas import tpu_sc as plsc

B = 16384
HID = 64
VOCAB = 1000000
SPLIT = 524288
PBLK = 2048
NC = 2
NS = 16
NW = NC * NS
BPW = B // NW
CH = 128
NCH = BPW // CH


def _pack_body(lo_ref, hi_ref, o_ref):
    lo_t = jnp.swapaxes(lo_ref[...], 0, 1)
    hi_t = jnp.swapaxes(hi_ref[...], 0, 1)
    o_ref[...] = jnp.concatenate([lo_t, hi_t], axis=1)


def _pack(tbl_t):
    grid = SPLIT // PBLK
    return pl.pallas_call(
        _pack_body,
        grid=(grid,),
        in_specs=[
            pl.BlockSpec((HID, PBLK), lambda g: (0, g)),
            pl.BlockSpec((HID, PBLK),
                         lambda g: (0, jnp.minimum(g + SPLIT // PBLK,
                                                   VOCAB // PBLK))),
        ],
        out_specs=pl.BlockSpec((PBLK, 2 * HID), lambda g: (g, 0)),
        out_shape=jax.ShapeDtypeStruct((SPLIT, 2 * HID), jnp.float32),
    )(tbl_t, tbl_t)


def _sc_gather(uidx3, iidx3, upacked, ipacked):
    mesh = plsc.VectorSubcoreMesh(core_axis_name="c", subcore_axis_name="s")

    @functools.partial(
        pl.kernel,
        out_type=[
            jax.ShapeDtypeStruct((B, 2 * HID), jnp.float32),
            jax.ShapeDtypeStruct((B, 2 * HID), jnp.float32),
        ],
        mesh=mesh,
        scratch_types=[
            pltpu.VMEM((NCH, CH), jnp.int32),
            pltpu.VMEM((NCH, CH), jnp.int32),
            pltpu.VMEM((BPW, 2 * HID), jnp.float32),
            pltpu.SemaphoreType.DMA,
        ],
        compiler_params=pltpu.CompilerParams(use_tc_tiling_on_sc=False),
    )
    def k(uidx_hbm, iidx_hbm, ut_hbm, it_hbm, uout_hbm, iout_hbm,
          uidx_v, iidx_v, urows_v, sem):
        wid = lax.axis_index("s") * NC + lax.axis_index("c")
        base = wid * BPW
        pltpu.sync_copy(uidx_hbm.at[wid], uidx_v)
        pltpu.sync_copy(iidx_hbm.at[wid], iidx_v)
        for idx_v, tbl_hbm, out_hbm in (
                (uidx_v, ut_hbm, uout_hbm), (iidx_v, it_hbm, iout_hbm)):
            copies = []
            for j in range(NCH):
                copies.append(pltpu.async_copy(
                    tbl_hbm.at[idx_v.at[j]],
                    urows_v.at[pl.ds(j * CH, CH)], sem))
            for c in copies:
                c.wait()
            pltpu.sync_copy(urows_v, out_hbm.at[pl.ds(base, BPW)])

    return k(uidx3, iidx3, upacked, ipacked)


def _tc_mlp_body(u_ref, i_ref, um_ref, im_ref, w1a_ref, w1b_ref, b1_ref,
                 w2_ref, b2_ref, o_ref):
    um = um_ref[...]
    im = im_ref[...]
    u2 = u_ref[...]
    i2 = i_ref[...]
    u = u2[:, :HID] * (1.0 - um) + u2[:, HID:] * um
    i = i2[:, :HID] * (1.0 - im) + i2[:, HID:] * im
    h = u @ w1a_ref[...] + i @ w1b_ref[...] + b1_ref[...]
    h = jnp.maximum(h, 0.0)
    y = h @ w2_ref[...] + b2_ref[0, 0]
    o_ref[...] = 1.0 / (1.0 + jnp.exp(-y))


def _tc_mlp(u2, i2, um, im, W1, b1, W2, b2):
    blk = 2048
    grid = B // blk
    return pl.pallas_call(
        _tc_mlp_body,
        grid=(grid,),
        in_specs=[
            pl.BlockSpec((blk, 2 * HID), lambda g: (g, 0)),
            pl.BlockSpec((blk, 2 * HID), lambda g: (g, 0)),
            pl.BlockSpec((blk, 1), lambda g: (g, 0)),
            pl.BlockSpec((blk, 1), lambda g: (g, 0)),
            pl.BlockSpec((HID, HID), lambda g: (0, 0)),
            pl.BlockSpec((HID, HID), lambda g: (0, 0)),
            pl.BlockSpec((1, HID), lambda g: (0, 0)),
            pl.BlockSpec((HID, 1), lambda g: (0, 0)),
            pl.BlockSpec((1, 1), lambda g: (0, 0), memory_space=pltpu.SMEM),
        ],
        out_specs=pl.BlockSpec((blk, 1), lambda g: (g, 0)),
        out_shape=jax.ShapeDtypeStruct((B, 1), jnp.float32),
    )(u2, i2, um, im, W1[:HID], W1[HID:], b1.reshape(1, HID),
      W2, b2.reshape(1, 1))


def kernel(user_id, item_id, h, user_table, item_table, W1, b1, W2, b2):
    del h
    uid = user_id.astype(jnp.int32)
    iid = item_id.astype(jnp.int32)
    uhi = uid >= SPLIT
    ihi = iid >= SPLIT
    uidx = jnp.where(uhi, uid - SPLIT, uid).reshape(NW, NCH, CH)
    iidx = jnp.where(ihi, iid - SPLIT, iid).reshape(NW, NCH, CH)
    um = uhi.astype(jnp.float32).reshape(B, 1)
    im = ihi.astype(jnp.float32).reshape(B, 1)
    upacked = _pack(user_table.T)
    ipacked = _pack(item_table.T)
    u2, i2 = _sc_gather(uidx, iidx, upacked, ipacked)
    out = _tc_mlp(u2, i2, um, im, W1, b1, W2, b2)
    return out.reshape(B)


# speedup vs baseline: 2.3787x; 2.3787x over previous
import functools

import jax
import jax.numpy as jnp
from jax import lax
from jax.experimental import pallas as pl
from jax.experimental.pallas import tpu as pltpu
from jax.experimental.pallas import tpu_sc as plsc

B = 16384
HID = 64
VOCAB = 1000000
SPLIT = 262144
PBLK = 8192
NC = 2
NS = 16
NW = NC * NS
BPW = B // NW
CH = 128
NCH = BPW // CH


def _pack_pair(a, b):
    abits = jax.lax.bitcast_convert_type(
        jax.lax.convert_element_type(a, jnp.bfloat16), jnp.uint16)
    bbits = jax.lax.bitcast_convert_type(
        jax.lax.convert_element_type(b, jnp.bfloat16), jnp.uint16)
    return (abits.astype(jnp.uint32) << 16) | bbits.astype(jnp.uint32)


def _pack_body(w0_ref, w1_ref, w2_ref, w3_ref, o_ref):
    p01 = _pack_pair(w0_ref[...], w1_ref[...])
    p23 = _pack_pair(w2_ref[...], w3_ref[...])
    a = jnp.concatenate([p01, p23], axis=0)
    o_ref[...] = jnp.swapaxes(a, 0, 1)


def _pack(tbl_t):
    grid = SPLIT // PBLK
    nb = SPLIT // PBLK
    last = VOCAB // PBLK

    def win(q):
        return pl.BlockSpec(
            (HID, PBLK), lambda g, q=q: (0, jnp.minimum(g + q * nb, last)))

    return pl.pallas_call(
        _pack_body,
        grid=(grid,),
        in_specs=[win(0), win(1), win(2), win(3)],
        out_specs=pl.BlockSpec((PBLK, 2 * HID), lambda g: (g, 0)),
        out_shape=jax.ShapeDtypeStruct((SPLIT, 2 * HID), jnp.uint32),
    )(tbl_t, tbl_t, tbl_t, tbl_t)


def _sc_gather(uidx3, iidx3, upacked, ipacked):
    mesh = plsc.VectorSubcoreMesh(core_axis_name="c", subcore_axis_name="s")

    @functools.partial(
        pl.kernel,
        out_type=[
            jax.ShapeDtypeStruct((B, 2 * HID), jnp.uint32),
            jax.ShapeDtypeStruct((B, 2 * HID), jnp.uint32),
        ],
        mesh=mesh,
        scratch_types=[
            pltpu.VMEM((NCH, CH), jnp.int32),
            pltpu.VMEM((NCH, CH), jnp.int32),
            pltpu.VMEM((BPW, 2 * HID), jnp.uint32),
            pltpu.SemaphoreType.DMA,
        ],
        compiler_params=pltpu.CompilerParams(use_tc_tiling_on_sc=False),
    )
    def k(uidx_hbm, iidx_hbm, ut_hbm, it_hbm, uout_hbm, iout_hbm,
          uidx_v, iidx_v, urows_v, sem):
        wid = lax.axis_index("s") * NC + lax.axis_index("c")
        base = wid * BPW
        pltpu.sync_copy(uidx_hbm.at[wid], uidx_v)
        pltpu.sync_copy(iidx_hbm.at[wid], iidx_v)
        for idx_v, tbl_hbm, out_hbm in (
                (uidx_v, ut_hbm, uout_hbm), (iidx_v, it_hbm, iout_hbm)):
            copies = []
            for j in range(NCH):
                copies.append(pltpu.async_copy(
                    tbl_hbm.at[idx_v.at[j]],
                    urows_v.at[pl.ds(j * CH, CH)], sem))
            for c in copies:
                c.wait()
            pltpu.sync_copy(urows_v, out_hbm.at[pl.ds(base, BPW)])

    return k(uidx3, iidx3, upacked, ipacked)


def _unpack_rows(w2col, q):
    sel = jnp.where(q >= 2, w2col[:, HID:], w2col[:, :HID])
    hi = jax.lax.bitcast_convert_type(sel & jnp.uint32(0xFFFF0000),
                                      jnp.float32)
    lo = jax.lax.bitcast_convert_type(sel << 16, jnp.float32)
    return jnp.where((q & 1) == 1, lo, hi)


def _tc_mlp_body(u_ref, i_ref, uq_ref, iq_ref, w1a_ref, w1b_ref, b1_ref,
                 w2_ref, b2_ref, o_ref):
    u = _unpack_rows(u_ref[...], uq_ref[...])
    i = _unpack_rows(i_ref[...], iq_ref[...])
    h = u @ w1a_ref[...] + i @ w1b_ref[...] + b1_ref[...]
    h = jnp.maximum(h, 0.0)
    y = h @ w2_ref[...] + b2_ref[0, 0]
    o_ref[...] = 1.0 / (1.0 + jnp.exp(-y))


def _tc_mlp(u2, i2, uq, iq, W1, b1, W2, b2):
    blk = 2048
    grid = B // blk
    return pl.pallas_call(
        _tc_mlp_body,
        grid=(grid,),
        in_specs=[
            pl.BlockSpec((blk, 2 * HID), lambda g: (g, 0)),
            pl.BlockSpec((blk, 2 * HID), lambda g: (g, 0)),
            pl.BlockSpec((blk, 1), lambda g: (g, 0)),
            pl.BlockSpec((blk, 1), lambda g: (g, 0)),
            pl.BlockSpec((HID, HID), lambda g: (0, 0)),
            pl.BlockSpec((HID, HID), lambda g: (0, 0)),
            pl.BlockSpec((1, HID), lambda g: (0, 0)),
            pl.BlockSpec((HID, 1), lambda g: (0, 0)),
            pl.BlockSpec((1, 1), lambda g: (0, 0), memory_space=pltpu.SMEM),
        ],
        out_specs=pl.BlockSpec((blk, 1), lambda g: (g, 0)),
        out_shape=jax.ShapeDtypeStruct((B, 1), jnp.float32),
    )(u2, i2, uq, iq, W1[:HID], W1[HID:], b1.reshape(1, HID),
      W2, b2.reshape(1, 1))


def kernel(user_id, item_id, h, user_table, item_table, W1, b1, W2, b2):
    del h
    uid = user_id.astype(jnp.int32)
    iid = item_id.astype(jnp.int32)
    uidx = (uid % SPLIT).reshape(NW, NCH, CH)
    iidx = (iid % SPLIT).reshape(NW, NCH, CH)
    uq = (uid // SPLIT).reshape(B, 1)
    iq = (iid // SPLIT).reshape(B, 1)
    upacked = _pack(user_table.T)
    ipacked = _pack(item_table.T)
    u2, i2 = _sc_gather(uidx, iidx, upacked, ipacked)
    out = _tc_mlp(u2, i2, uq, iq, W1, b1, W2, b2)
    return out.reshape(B)


# speedup vs baseline: 2.3856x; 1.0029x over previous
import functools

import jax
import jax.numpy as jnp
from jax import lax
from jax.experimental import pallas as pl
from jax.experimental.pallas import tpu as pltpu
from jax.experimental.pallas import tpu_sc as plsc

B = 16384
HID = 64
VOCAB = 1000000
SPLIT = 262144
PBLK = 16384
NC = 2
NS = 16
NW = NC * NS
BPW = B // NW
CH = 128
NCH = BPW // CH


def _pack_pair(a, b):
    abits = jax.lax.bitcast_convert_type(
        jax.lax.convert_element_type(a, jnp.bfloat16), jnp.uint16)
    bbits = jax.lax.bitcast_convert_type(
        jax.lax.convert_element_type(b, jnp.bfloat16), jnp.uint16)
    return (abits.astype(jnp.uint32) << 16) | bbits.astype(jnp.uint32)


def _pack_body(w0_ref, w1_ref, w2_ref, w3_ref, o_ref):
    p01 = _pack_pair(w0_ref[...], w1_ref[...])
    p23 = _pack_pair(w2_ref[...], w3_ref[...])
    a = jnp.concatenate([p01, p23], axis=0)
    o_ref[...] = jnp.swapaxes(a, 0, 1)


def _pack(tbl_t):
    grid = SPLIT // PBLK
    nb = SPLIT // PBLK
    last = VOCAB // PBLK

    def win(q):
        return pl.BlockSpec(
            (HID, PBLK), lambda g, q=q: (0, jnp.minimum(g + q * nb, last)))

    return pl.pallas_call(
        _pack_body,
        grid=(grid,),
        in_specs=[win(0), win(1), win(2), win(3)],
        out_specs=pl.BlockSpec((PBLK, 2 * HID), lambda g: (g, 0)),
        out_shape=jax.ShapeDtypeStruct((SPLIT, 2 * HID), jnp.uint32),
    )(tbl_t, tbl_t, tbl_t, tbl_t)


def _sc_gather(uidx3, iidx3, upacked, ipacked):
    mesh = plsc.VectorSubcoreMesh(core_axis_name="c", subcore_axis_name="s")

    @functools.partial(
        pl.kernel,
        out_type=[
            jax.ShapeDtypeStruct((B, 2 * HID), jnp.uint32),
            jax.ShapeDtypeStruct((B, 2 * HID), jnp.uint32),
        ],
        mesh=mesh,
        scratch_types=[
            pltpu.VMEM((NCH, CH), jnp.int32),
            pltpu.VMEM((NCH, CH), jnp.int32),
            pltpu.VMEM((BPW, 2 * HID), jnp.uint32),
            pltpu.SemaphoreType.DMA,
        ],
        compiler_params=pltpu.CompilerParams(use_tc_tiling_on_sc=False),
    )
    def k(uidx_hbm, iidx_hbm, ut_hbm, it_hbm, uout_hbm, iout_hbm,
          uidx_v, iidx_v, urows_v, sem):
        wid = lax.axis_index("s") * NC + lax.axis_index("c")
        base = wid * BPW
        pltpu.sync_copy(uidx_hbm.at[wid], uidx_v)
        pltpu.sync_copy(iidx_hbm.at[wid], iidx_v)
        for idx_v, tbl_hbm, out_hbm in (
                (uidx_v, ut_hbm, uout_hbm), (iidx_v, it_hbm, iout_hbm)):
            copies = []
            for j in range(NCH):
                copies.append(pltpu.async_copy(
                    tbl_hbm.at[idx_v.at[j]],
                    urows_v.at[pl.ds(j * CH, CH)], sem))
            for c in copies:
                c.wait()
            pltpu.sync_copy(urows_v, out_hbm.at[pl.ds(base, BPW)])

    return k(uidx3, iidx3, upacked, ipacked)


def _unpack_rows(w2col, q):
    sel = jnp.where(q >= 2, w2col[:, HID:], w2col[:, :HID])
    hi = jax.lax.bitcast_convert_type(sel & jnp.uint32(0xFFFF0000),
                                      jnp.float32)
    lo = jax.lax.bitcast_convert_type(sel << 16, jnp.float32)
    return jnp.where((q & 1) == 1, lo, hi)


def _tc_mlp_body(u_ref, i_ref, uq_ref, iq_ref, w1a_ref, w1b_ref, b1_ref,
                 w2_ref, b2_ref, o_ref):
    u = _unpack_rows(u_ref[...], uq_ref[...])
    i = _unpack_rows(i_ref[...], iq_ref[...])
    h = u @ w1a_ref[...] + i @ w1b_ref[...] + b1_ref[...]
    h = jnp.maximum(h, 0.0)
    y = h @ w2_ref[...] + b2_ref[0, 0]
    o_ref[...] = 1.0 / (1.0 + jnp.exp(-y))


def _tc_mlp(u2, i2, uq, iq, W1, b1, W2, b2):
    blk = 2048
    grid = B // blk
    return pl.pallas_call(
        _tc_mlp_body,
        grid=(grid,),
        in_specs=[
            pl.BlockSpec((blk, 2 * HID), lambda g: (g, 0)),
            pl.BlockSpec((blk, 2 * HID), lambda g: (g, 0)),
            pl.BlockSpec((blk, 1), lambda g: (g, 0)),
            pl.BlockSpec((blk, 1), lambda g: (g, 0)),
            pl.BlockSpec((HID, HID), lambda g: (0, 0)),
            pl.BlockSpec((HID, HID), lambda g: (0, 0)),
            pl.BlockSpec((1, HID), lambda g: (0, 0)),
            pl.BlockSpec((HID, 1), lambda g: (0, 0)),
            pl.BlockSpec((1, 1), lambda g: (0, 0), memory_space=pltpu.SMEM),
        ],
        out_specs=pl.BlockSpec((blk, 1), lambda g: (g, 0)),
        out_shape=jax.ShapeDtypeStruct((B, 1), jnp.float32),
    )(u2, i2, uq, iq, W1[:HID], W1[HID:], b1.reshape(1, HID),
      W2, b2.reshape(1, 1))


def kernel(user_id, item_id, h, user_table, item_table, W1, b1, W2, b2):
    del h
    uid = user_id.astype(jnp.int32)
    iid = item_id.astype(jnp.int32)
    uidx = (uid % SPLIT).reshape(NW, NCH, CH)
    iidx = (iid % SPLIT).reshape(NW, NCH, CH)
    uq = (uid // SPLIT).reshape(B, 1)
    iq = (iid // SPLIT).reshape(B, 1)
    upacked = _pack(user_table.T)
    ipacked = _pack(item_table.T)
    u2, i2 = _sc_gather(uidx, iidx, upacked, ipacked)
    out = _tc_mlp(u2, i2, uq, iq, W1, b1, W2, b2)
    return out.reshape(B)
